# baseline (speedup 1.0000x reference)
import functools

import jax
import jax.numpy as jnp
import numpy as np
from jax import lax
from jax.experimental import pallas as pl
from jax.experimental.pallas import tpu as pltpu
from jax.experimental.pallas import tpu_sc as plsc

N = 10000
NHALF = N // 2
NF = 128
NFF = 2 * NF
E = 320000
NTILES = 16
NCORES = 2
SUB = 64
GRAN = 1024
NPIECE = GRAN // SUB
EPAD = 327680
GRANS_MAX = EPAD // (NTILES * GRAN)
ROWS_PT = 624
ROWS_TAIL = N - NTILES * ROWS_PT
WB_PT = 312
WB_TAIL = NHALF - NTILES * WB_PT

_HIGH = jax.lax.Precision.HIGHEST

_SIGMA = np.empty((NF,), dtype=np.int32)
for _g in range(NF // 32):
    for _k in range(16):
        _SIGMA[_g * 32 + 2 * _k] = _g * 32 + _k
        _SIGMA[_g * 32 + 2 * _k + 1] = _g * 32 + 16 + _k


def _mm_body(x_ref, w_ref, o_ref):
    o_ref[...] = jax.lax.dot_general(
        x_ref[...], w_ref[...], (((1,), (0,)), ((), ())),
        preferred_element_type=jnp.float32,
        precision=_HIGH).astype(jnp.bfloat16)


def _dense_fts(seqs, Wp):
    BLK = 2000
    NB = N // BLK
    return pl.pallas_call(
        _mm_body,
        grid=(NB, 2),
        in_specs=[pl.BlockSpec((BLK, NF), lambda i, j: (j * NB + i, 0)),
                  pl.BlockSpec((NF, NF), lambda i, j: (0, 0))],
        out_specs=pl.BlockSpec((BLK, NF), lambda i, j: (i, j)),
        out_shape=jax.ShapeDtypeStruct((N, NFF), jnp.bfloat16),
    )(seqs, Wp)


def _sc_spmm(table, cols_a, rows_a, ew_a, counts, zeros):
    mesh = plsc.VectorSubcoreMesh(core_axis_name="c", subcore_axis_name="s")

    @functools.partial(
        pl.kernel,
        out_type=jax.ShapeDtypeStruct((NCORES, N, NF), jnp.float32),
        mesh=mesh,
        scratch_types=[
            pltpu.VMEM_SHARED((N, NF), jnp.float32),
            pltpu.VMEM((NPIECE, SUB), jnp.int32),
            pltpu.VMEM((NPIECE, SUB), jnp.int32),
            pltpu.VMEM((NPIECE, SUB), jnp.int32),
            pltpu.VMEM((GRAN // 16, 16), jnp.float32),
            pltpu.VMEM((16,), jnp.int32),
            pltpu.VMEM((SUB, NFF), jnp.bfloat16),
            pltpu.VMEM((SUB, NFF), jnp.bfloat16),
            pltpu.VMEM((SUB, NF), jnp.float32),
            pltpu.VMEM((SUB, NF), jnp.float32),
            pltpu.SemaphoreType.DMA,
            pltpu.SemaphoreType.DMA,
        ],
        compiler_params=pltpu.CompilerParams(use_tc_tiling_on_sc=False,
                                             needs_layout_passes=False),
    )
    def k(table_hbm, cols_hbm, rows_hbm, ew_hbm, counts_hbm, zeros_hbm,
          out_hbm, acc, colv, rowlo, rowhi, ewv, cntv, msgs0, msgs1,
          scaled_a, scaled_b, sem0, sem1):
        c = lax.axis_index("c")
        s = lax.axis_index("s")
        bufs = (msgs0, msgs1)
        sems = (sem0, sem1)

        pltpu.sync_copy(counts_hbm.at[c], cntv)
        T = jnp.max(cntv[...])

        pltpu.sync_copy(zeros_hbm.at[pl.ds(0, ROWS_PT)],
                        acc.at[pl.ds(s * ROWS_PT, ROWS_PT)])

        @pl.when(s == NTILES - 1)
        def _():
            pltpu.sync_copy(zeros_hbm.at[pl.ds(0, ROWS_TAIL)],
                            acc.at[pl.ds(NTILES * ROWS_PT, ROWS_TAIL)])

        plsc.subcore_barrier()

        def step_body(kk, carry):
            g = s * T + kk
            off = g * NPIECE
            offw = g * (GRAN // 16)
            pltpu.sync_copy(cols_hbm.at[c, pl.ds(off, NPIECE)], colv)
            pltpu.sync_copy(rows_hbm.at[c, pl.ds(off, NPIECE)], rowlo)
            pltpu.sync_copy(ew_hbm.at[c, pl.ds(offw, GRAN // 16)], ewv)
            for r in range(NPIECE):
                for j in range(SUB // 16):
                    sl = pl.ds(j * 16, 16)
                    rowhi[r, sl] = rowlo[r, sl] + NHALF
            pltpu.async_copy(table_hbm.at[colv.at[0]], msgs0, sem0)

            def piece_pair(pc, carry2):
                for b in range(2):
                    p = pc + b
                    buf, sem = bufs[b], sems[b]
                    nbuf, nsem = bufs[1 - b], sems[1 - b]
                    pltpu.make_async_copy(table_hbm.at[colv.at[p]], buf,
                                          sem).wait()

                    @pl.when(p < NPIECE - 1)
                    def _():
                        pltpu.async_copy(table_hbm.at[colv.at[p + 1]],
                                         nbuf, nsem)

                    def mul_body(g2, carry3):
                        w = ewv[p * (SUB // 16) + g2]
                        for e16 in range(16):
                            wb = jnp.broadcast_to(w[e16], (16,))
                            e = g2 * 16 + e16
                            for f in range(NFF // 32):
                                xi = plsc.bitcast(
                                    buf[e, pl.ds(f * 32, 32)], jnp.int32)
                                lo = plsc.bitcast(xi << 16, jnp.float32)
                                hi = plsc.bitcast(
                                    xi & jnp.int32(-65536), jnp.float32)
                                dst = scaled_a if f < NF // 32 else scaled_b
                                fo = (f % (NF // 32)) * 32
                                dst[e, pl.ds(fo, 16)] = lo * wb
                                dst[e, pl.ds(fo + 16, 16)] = hi * wb
                        return carry3

                    lax.fori_loop(0, SUB // 16, mul_body, 0, unroll=False)
                    pltpu.sync_copy(scaled_a, acc.at[rowlo.at[p]], add=True)
                    pltpu.sync_copy(scaled_b, acc.at[rowhi.at[p]], add=True)
                return carry2

            lax.fori_loop(0, NPIECE // 2,
                          lambda i, cc: piece_pair(i * 2, cc),
                          0, unroll=False)
            return carry

        lax.fori_loop(0, T, step_body, 0, unroll=False)
        plsc.subcore_barrier()
        for l in range(2):
            pltpu.sync_copy(
                acc.at[pl.ds(l * NHALF + s * WB_PT, WB_PT)],
                out_hbm.at[l, pl.ds(c * NHALF + s * WB_PT, WB_PT)])

        @pl.when(s == NTILES - 1)
        def _():
            for l in range(2):
                pltpu.sync_copy(
                    acc.at[pl.ds(l * NHALF + NTILES * WB_PT, WB_TAIL)],
                    out_hbm.at[l, pl.ds(c * NHALF + NTILES * WB_PT, WB_TAIL)])

    return k(table, cols_a, rows_a, ew_a, counts, zeros)


FBLK = 2000


def _csum_body(agg1_ref, b_ref, mskT_ref, out_ref):
    h1 = jnp.maximum(agg1_ref[...] + b_ref[...], 0.0)
    part = jax.lax.dot_general(h1, mskT_ref[...], (((0,), (0,)), ((), ())),
                               preferred_element_type=jnp.float32,
                               precision=_HIGH)

    @pl.when(pl.program_id(0) == 0)
    def _():
        out_ref[...] = part

    @pl.when(pl.program_id(0) > 0)
    def _():
        out_ref[...] += part


def _logits_body(csum_ref, mskT_ref, wd_ref, agg_ref, b_ref, sb_ref, bd_ref,
                 out_ref):
    cvec = csum_ref[...] / jnp.sum(mskT_ref[...])
    cvec = 1.0 / (1.0 + jnp.exp(-cvec))
    u = jax.lax.dot_general(wd_ref[...], cvec, (((1,), (0,)), ((), ())),
                            preferred_element_type=jnp.float32,
                            precision=_HIGH)
    h = jnp.maximum(agg_ref[...] + b_ref[...], 0.0)
    s = jax.lax.dot_general(h, u, (((1,), (0,)), ((), ())),
                            preferred_element_type=jnp.float32,
                            precision=_HIGH)
    out_ref[...] = s + bd_ref[0, 0] + sb_ref[...]


def _final(agg, b_gcn, mskT, wd, sb, bd):
    csum = pl.pallas_call(
        _csum_body,
        grid=(N // FBLK,),
        in_specs=[pl.BlockSpec((FBLK, NF), lambda i: (i, 0)),
                  pl.BlockSpec((1, NF), lambda i: (0, 0)),
                  pl.BlockSpec((FBLK, 1), lambda i: (i, 0))],
        out_specs=pl.BlockSpec((NF, 1), lambda i: (0, 0)),
        out_shape=jax.ShapeDtypeStruct((NF, 1), jnp.float32),
    )(agg[:N], b_gcn, mskT)
    return pl.pallas_call(
        _logits_body,
        grid=(2 * N // FBLK,),
        in_specs=[pl.BlockSpec((NF, 1), lambda i: (0, 0)),
                  pl.BlockSpec((N, 1), lambda i: (0, 0)),
                  pl.BlockSpec((NF, NF), lambda i: (0, 0)),
                  pl.BlockSpec((FBLK, NF), lambda i: (i, 0)),
                  pl.BlockSpec((1, NF), lambda i: (0, 0)),
                  pl.BlockSpec((FBLK, 1), lambda i: (i, 0)),
                  pl.BlockSpec((1, 1), lambda i: (0, 0))],
        out_specs=pl.BlockSpec((FBLK, 1), lambda i: (i, 0)),
        out_shape=jax.ShapeDtypeStruct((2 * N, 1), jnp.float32),
    )(csum, mskT, wd, agg, b_gcn, sb, bd)


def kernel(seq1, seq2, edge_index, edge_weight, msk, samp_bias1, samp_bias2,
           W_gcn, b_gcn, W_disc, b_disc):
    seqs = jnp.concatenate([seq1[0], seq2[0]], axis=0)
    Wp = W_gcn[:, jnp.asarray(_SIGMA)]
    table = _dense_fts(seqs, Wp)

    row = edge_index[0]
    col = edge_index[1]
    ew = edge_weight
    m0 = row < NHALF
    r0 = jnp.cumsum(m0.astype(jnp.int32)) - 1
    r1 = jnp.cumsum((~m0).astype(jnp.int32)) - 1
    pos = jnp.where(m0, r0, EPAD + r1)
    row_local = jnp.where(m0, row, row - NHALF)
    colP = jnp.zeros((2 * EPAD,), jnp.int32).at[pos].set(
        col, unique_indices=True)
    rowP = jnp.zeros((2 * EPAD,), jnp.int32).at[pos].set(
        row_local, unique_indices=True)
    ewP = jnp.zeros((2 * EPAD,), jnp.float32).at[pos].set(
        ew, unique_indices=True)
    cols_a = colP.reshape(NCORES, EPAD // SUB, SUB)
    rows_a = rowP.reshape(NCORES, EPAD // SUB, SUB)
    ew_a = ewP.reshape(NCORES, EPAD // 16, 16)
    e0 = jnp.sum(m0.astype(jnp.int32))
    per = NTILES * GRAN
    t0 = (e0 + per - 1) // per
    t1 = ((E - e0) + per - 1) // per
    counts = (jnp.stack([t0, t1]).astype(jnp.int32)[:, None]
              * jnp.ones((1, NTILES), jnp.int32))
    zeros = jnp.zeros((ROWS_PT + ROWS_TAIL, NF), jnp.float32)

    agg = _sc_spmm(table, cols_a, rows_a, ew_a, counts, zeros)

    sb = jnp.concatenate([samp_bias1, samp_bias2], axis=1).reshape(2 * N, 1)
    out = _final(agg.reshape(2 * N, NF), b_gcn.reshape(1, NF),
                 msk.reshape(N, 1), W_disc, sb, b_disc.reshape(1, 1))
    return out.reshape(1, 2 * N)


# speedup vs baseline: 3.0774x; 3.0774x over previous
import functools

import jax
import jax.numpy as jnp
import numpy as np
from jax import lax
from jax.experimental import pallas as pl
from jax.experimental.pallas import tpu as pltpu
from jax.experimental.pallas import tpu_sc as plsc

N = 10000
NHALF = N // 2
NF = 128
NFF = 2 * NF
E = 320000
NTILES = 16
NCORES = 2
SUB = 64
GRAN = 1024
NPIECE = GRAN // SUB
EPAD = 327680
GRANS_MAX = EPAD // (NTILES * GRAN)
ROWS_PT = 624
ROWS_TAIL = N - NTILES * ROWS_PT
WB_PT = 312
WB_TAIL = NHALF - NTILES * WB_PT

_HIGH = jax.lax.Precision.HIGHEST

_SIGMA = np.empty((NF,), dtype=np.int32)
for _g in range(NF // 32):
    for _k in range(16):
        _SIGMA[_g * 32 + 2 * _k] = _g * 32 + _k
        _SIGMA[_g * 32 + 2 * _k + 1] = _g * 32 + 16 + _k


def _mm_body(x_ref, w_ref, o_ref):
    o_ref[...] = jax.lax.dot_general(
        x_ref[...], w_ref[...], (((1,), (0,)), ((), ())),
        preferred_element_type=jnp.float32,
        precision=_HIGH).astype(jnp.bfloat16)


def _dense_fts(seqs, Wp):
    BLK = 2000
    NB = N // BLK
    return pl.pallas_call(
        _mm_body,
        grid=(NB, 2),
        in_specs=[pl.BlockSpec((BLK, NF), lambda i, j: (j * NB + i, 0)),
                  pl.BlockSpec((NF, NF), lambda i, j: (0, 0))],
        out_specs=pl.BlockSpec((BLK, NF), lambda i, j: (i, j)),
        out_shape=jax.ShapeDtypeStruct((N, NFF), jnp.bfloat16),
    )(seqs, Wp)


def _sc_spmm(table, cols_a, rows_a, ew_a, zeros):
    mesh = plsc.VectorSubcoreMesh(core_axis_name="c", subcore_axis_name="s")

    @functools.partial(
        pl.kernel,
        out_type=jax.ShapeDtypeStruct((NCORES, N, NF), jnp.float32),
        mesh=mesh,
        scratch_types=[
            pltpu.VMEM_SHARED((N, NF), jnp.float32),
            pltpu.VMEM((NPIECE, SUB), jnp.int32),
            pltpu.VMEM((NPIECE, SUB), jnp.int32),
            pltpu.VMEM((NPIECE, SUB), jnp.int32),
            pltpu.VMEM((GRAN // 16, 16), jnp.float32),
            pltpu.VMEM((SUB, NFF), jnp.bfloat16),
            pltpu.VMEM((SUB, NFF), jnp.bfloat16),
            pltpu.VMEM((SUB, NF), jnp.float32),
            pltpu.VMEM((SUB, NF), jnp.float32),
            pltpu.SemaphoreType.DMA,
            pltpu.SemaphoreType.DMA,
        ],
        compiler_params=pltpu.CompilerParams(use_tc_tiling_on_sc=False,
                                             needs_layout_passes=False),
    )
    def k(table_hbm, cols_hbm, rows_hbm, ew_hbm, zeros_hbm,
          out_hbm, acc, colv, rowlo, rowhi, ewv, msgs0, msgs1,
          scaled_a, scaled_b, sem0, sem1):
        c = lax.axis_index("c")
        s = lax.axis_index("s")
        bufs = (msgs0, msgs1)
        sems = (sem0, sem1)

        pltpu.sync_copy(zeros_hbm.at[pl.ds(0, ROWS_PT)],
                        acc.at[pl.ds(s * ROWS_PT, ROWS_PT)])

        @pl.when(s == NTILES - 1)
        def _():
            pltpu.sync_copy(zeros_hbm.at[pl.ds(0, ROWS_TAIL)],
                            acc.at[pl.ds(NTILES * ROWS_PT, ROWS_TAIL)])

        plsc.subcore_barrier()

        def _gidx(p):
            return plsc.Indices(colv.at[p], ignored_value=-1)

        def step_body(kk, carry):
            g = s * (EPAD // (NTILES * GRAN)) + kk
            off = g * NPIECE
            offw = g * (GRAN // 16)
            pltpu.sync_copy(cols_hbm.at[c, pl.ds(off, NPIECE)], colv)
            pltpu.sync_copy(rows_hbm.at[c, pl.ds(off, NPIECE)], rowlo)
            pltpu.sync_copy(ew_hbm.at[pl.ds(offw, GRAN // 16)], ewv)
            for r in range(NPIECE):
                for j in range(SUB // 16):
                    sl = pl.ds(j * 16, 16)
                    lo = rowlo[r, sl]
                    rowhi[r, sl] = jnp.where(lo < 0, lo, lo + NHALF)
            pltpu.async_copy(table_hbm.at[_gidx(0)], msgs0, sem0)

            def piece_pair(pc, carry2):
                for b in range(2):
                    p = pc + b
                    buf, sem = bufs[b], sems[b]
                    nbuf, nsem = bufs[1 - b], sems[1 - b]
                    pltpu.make_async_copy(table_hbm.at[_gidx(p)], buf,
                                          sem).wait()

                    @pl.when(p < NPIECE - 1)
                    def _():
                        pltpu.async_copy(table_hbm.at[_gidx(p + 1)],
                                         nbuf, nsem)

                    def mul_body(g2, carry3):
                        w = ewv[p * (SUB // 16) + g2]
                        for e16 in range(16):
                            wb = jnp.broadcast_to(w[e16], (16,))
                            e = g2 * 16 + e16
                            for f in range(NFF // 32):
                                xi = plsc.bitcast(
                                    buf[e, pl.ds(f * 32, 32)], jnp.int32)
                                lo = plsc.bitcast(xi << 16, jnp.float32)
                                hi = plsc.bitcast(
                                    xi & jnp.int32(-65536), jnp.float32)
                                dst = scaled_a if f < NF // 32 else scaled_b
                                fo = (f % (NF // 32)) * 32
                                dst[e, pl.ds(fo, 16)] = lo * wb
                                dst[e, pl.ds(fo + 16, 16)] = hi * wb
                        return carry3

                    lax.fori_loop(0, SUB // 16, mul_body, 0, unroll=False)
                    pltpu.sync_copy(
                        scaled_a,
                        acc.at[plsc.Indices(rowlo.at[p], ignored_value=-1)],
                        add=True)
                    pltpu.sync_copy(
                        scaled_b,
                        acc.at[plsc.Indices(rowhi.at[p], ignored_value=-1)],
                        add=True)
                return carry2

            lax.fori_loop(0, NPIECE // 2,
                          lambda i, cc: piece_pair(i * 2, cc),
                          0, unroll=False)
            return carry

        lax.fori_loop(0, EPAD // (NTILES * GRAN), step_body, 0, unroll=False)
        plsc.subcore_barrier()
        for l in range(2):
            pltpu.sync_copy(
                acc.at[pl.ds(l * NHALF + s * WB_PT, WB_PT)],
                out_hbm.at[l, pl.ds(c * NHALF + s * WB_PT, WB_PT)])

        @pl.when(s == NTILES - 1)
        def _():
            for l in range(2):
                pltpu.sync_copy(
                    acc.at[pl.ds(l * NHALF + NTILES * WB_PT, WB_TAIL)],
                    out_hbm.at[l, pl.ds(c * NHALF + NTILES * WB_PT, WB_TAIL)])

    return k(table, cols_a, rows_a, ew_a, zeros)


FBLK = 2000


def _csum_body(agg1_ref, b_ref, mskT_ref, out_ref):
    h1 = jnp.maximum(agg1_ref[...] + b_ref[...], 0.0)
    part = jax.lax.dot_general(h1, mskT_ref[...], (((0,), (0,)), ((), ())),
                               preferred_element_type=jnp.float32,
                               precision=_HIGH)

    @pl.when(pl.program_id(0) == 0)
    def _():
        out_ref[...] = part

    @pl.when(pl.program_id(0) > 0)
    def _():
        out_ref[...] += part


def _logits_body(csum_ref, mskT_ref, wd_ref, agg_ref, b_ref, sb_ref, bd_ref,
                 out_ref):
    cvec = csum_ref[...] / jnp.sum(mskT_ref[...])
    cvec = 1.0 / (1.0 + jnp.exp(-cvec))
    u = jax.lax.dot_general(wd_ref[...], cvec, (((1,), (0,)), ((), ())),
                            preferred_element_type=jnp.float32,
                            precision=_HIGH)
    h = jnp.maximum(agg_ref[...] + b_ref[...], 0.0)
    s = jax.lax.dot_general(h, u, (((1,), (0,)), ((), ())),
                            preferred_element_type=jnp.float32,
                            precision=_HIGH)
    out_ref[...] = s + bd_ref[0, 0] + sb_ref[...]


def _final(agg, b_gcn, mskT, wd, sb, bd):
    csum = pl.pallas_call(
        _csum_body,
        grid=(N // FBLK,),
        in_specs=[pl.BlockSpec((FBLK, NF), lambda i: (i, 0)),
                  pl.BlockSpec((1, NF), lambda i: (0, 0)),
                  pl.BlockSpec((FBLK, 1), lambda i: (i, 0))],
        out_specs=pl.BlockSpec((NF, 1), lambda i: (0, 0)),
        out_shape=jax.ShapeDtypeStruct((NF, 1), jnp.float32),
    )(agg[:N], b_gcn, mskT)
    return pl.pallas_call(
        _logits_body,
        grid=(2 * N // FBLK,),
        in_specs=[pl.BlockSpec((NF, 1), lambda i: (0, 0)),
                  pl.BlockSpec((N, 1), lambda i: (0, 0)),
                  pl.BlockSpec((NF, NF), lambda i: (0, 0)),
                  pl.BlockSpec((FBLK, NF), lambda i: (i, 0)),
                  pl.BlockSpec((1, NF), lambda i: (0, 0)),
                  pl.BlockSpec((FBLK, 1), lambda i: (i, 0)),
                  pl.BlockSpec((1, 1), lambda i: (0, 0))],
        out_specs=pl.BlockSpec((FBLK, 1), lambda i: (i, 0)),
        out_shape=jax.ShapeDtypeStruct((2 * N, 1), jnp.float32),
    )(csum, mskT, wd, agg, b_gcn, sb, bd)


def kernel(seq1, seq2, edge_index, edge_weight, msk, samp_bias1, samp_bias2,
           W_gcn, b_gcn, W_disc, b_disc):
    seqs = jnp.concatenate([seq1[0], seq2[0]], axis=0)
    Wp = W_gcn[:, jnp.asarray(_SIGMA)]
    table = _dense_fts(seqs, Wp)

    row = edge_index[0]
    col = edge_index[1]
    pad = EPAD - E
    sent_pad = jnp.full((pad,), -1, jnp.int32)
    m0 = row < NHALF
    col0 = jnp.concatenate([jnp.where(m0, col, -1), sent_pad])
    col1 = jnp.concatenate([jnp.where(m0, -1, col), sent_pad])
    row0 = jnp.concatenate([jnp.where(m0, row, -1), sent_pad])
    row1 = jnp.concatenate([jnp.where(m0, -1, row - NHALF), sent_pad])
    cols_a = jnp.stack([col0, col1]).reshape(NCORES, EPAD // SUB, SUB)
    rows_a = jnp.stack([row0, row1]).reshape(NCORES, EPAD // SUB, SUB)
    ew_a = jnp.concatenate([edge_weight, jnp.zeros((pad,), jnp.float32)]
                           ).reshape(EPAD // 16, 16)
    zeros = jnp.zeros((ROWS_PT + ROWS_TAIL, NF), jnp.float32)

    agg = _sc_spmm(table, cols_a, rows_a, ew_a, zeros)

    sb = jnp.concatenate([samp_bias1, samp_bias2], axis=1).reshape(2 * N, 1)
    out = _final(agg.reshape(2 * N, NF), b_gcn.reshape(1, NF),
                 msk.reshape(N, 1), W_disc, sb, b_disc.reshape(1, 1))
    return out.reshape(1, 2 * N)


# speedup vs baseline: 5.3571x; 1.7408x over previous
import functools

import jax
import jax.numpy as jnp
from jax import lax
from jax.experimental import pallas as pl
from jax.experimental.pallas import tpu as pltpu
from jax.experimental.pallas import tpu_sc as plsc

N = 10000
NF = 128
E = 320000
NTILES = 16
NCORES = 2
SUB = 128
GRAN = 1024
EPT = 20480
NSTEPS = EPT // GRAN
EPAD = EPT * NTILES
ROWS_PT = 624
ROWS_TAIL = N - NTILES * ROWS_PT

_HIGH = jax.lax.Precision.HIGHEST


def _mm_body(x_ref, w_ref, o_ref):
    o_ref[...] = jax.lax.dot_general(
        x_ref[...], w_ref[...], (((1,), (0,)), ((), ())),
        preferred_element_type=jnp.float32, precision=_HIGH)


def _dense_fts(seqs, W):
    BLK = 2000
    return pl.pallas_call(
        _mm_body,
        grid=(2 * N // BLK,),
        in_specs=[pl.BlockSpec((BLK, NF), lambda i: (i, 0)),
                  pl.BlockSpec((NF, NF), lambda i: (0, 0))],
        out_specs=pl.BlockSpec((BLK, NF), lambda i: (i, 0)),
        out_shape=jax.ShapeDtypeStruct((2 * N, NF), jnp.float32),
    )(seqs, W)


def _sc_spmm(fts, cols2, rows_idx, ew2, zeros):
    mesh = plsc.VectorSubcoreMesh(core_axis_name="c", subcore_axis_name="s")

    @functools.partial(
        pl.kernel,
        out_type=jax.ShapeDtypeStruct((NCORES, N, NF), jnp.float32),
        mesh=mesh,
        scratch_types=[
            pltpu.VMEM_SHARED((N, NF), jnp.float32),
            pltpu.VMEM((GRAN // SUB, SUB), jnp.int32),
            pltpu.VMEM((GRAN // SUB, SUB), jnp.int32),
            pltpu.VMEM((GRAN // 16, 16), jnp.float32),
            pltpu.VMEM((SUB, NF), jnp.float32),
            pltpu.VMEM((SUB, NF), jnp.float32),
            pltpu.SemaphoreType.DMA,
            pltpu.SemaphoreType.DMA,
        ],
    )
    def k(fts_hbm, cols_hbm, rowsidx_hbm, ew_hbm, zeros_hbm, out_hbm,
          acc, colv, rowv, ewv, msgs0, msgs1, sem0, sem1):
        c = lax.axis_index("c")
        s = lax.axis_index("s")
        NPIECE = GRAN // SUB
        bufs = (msgs0, msgs1)
        sems = (sem0, sem1)

        pltpu.sync_copy(zeros_hbm.at[pl.ds(0, ROWS_PT)],
                        acc.at[pl.ds(s * ROWS_PT, ROWS_PT)])

        @pl.when(s == NTILES - 1)
        def _():
            pltpu.sync_copy(zeros_hbm.at[pl.ds(0, ROWS_TAIL)],
                            acc.at[pl.ds(NTILES * ROWS_PT, ROWS_TAIL)])

        plsc.subcore_barrier()

        def step_body(kk, carry):
            off = s * (EPT // SUB) + kk * NPIECE
            offw = s * (EPT // 16) + kk * (GRAN // 16)
            pltpu.sync_copy(cols_hbm.at[c, pl.ds(off, NPIECE)], colv)
            pltpu.sync_copy(rowsidx_hbm.at[pl.ds(off, NPIECE)], rowv)
            pltpu.sync_copy(ew_hbm.at[pl.ds(offw, GRAN // 16)], ewv)
            pltpu.async_copy(fts_hbm.at[colv.at[0]], msgs0, sem0)

            def piece_pair(pc, carry2):
                for b in range(2):
                    p = pc + b
                    buf, sem = bufs[b], sems[b]
                    nbuf, nsem = bufs[1 - b], sems[1 - b]
                    pltpu.make_async_copy(fts_hbm.at[colv.at[p]], buf,
                                          sem).wait()

                    @pl.when(p < NPIECE - 1)
                    def _():
                        pltpu.async_copy(fts_hbm.at[colv.at[p + 1]],
                                         nbuf, nsem)

                    def mul_body(g, carry3):
                        w = ewv[p * (SUB // 16) + g]
                        for e16 in range(16):
                            wb = jnp.broadcast_to(w[e16], (16,))
                            e = g * 16 + e16
                            for f in range(NF // 16):
                                sl = pl.ds(f * 16, 16)
                                buf[e, sl] = buf[e, sl] * wb
                        return carry3

                    lax.fori_loop(0, SUB // 16, mul_body, 0, unroll=False)
                    pltpu.sync_copy(buf, acc.at[rowv.at[p]], add=True)
                return carry2

            lax.fori_loop(0, NPIECE // 2,
                          lambda i, cc: piece_pair(i * 2, cc),
                          0, unroll=False)
            return carry

        lax.fori_loop(0, NSTEPS, step_body, 0, unroll=False)
        plsc.subcore_barrier()
        pltpu.sync_copy(acc.at[pl.ds(s * ROWS_PT, ROWS_PT)],
                        out_hbm.at[c, pl.ds(s * ROWS_PT, ROWS_PT)])

        @pl.when(s == NTILES - 1)
        def _():
            pltpu.sync_copy(acc.at[pl.ds(NTILES * ROWS_PT, ROWS_TAIL)],
                            out_hbm.at[c, pl.ds(NTILES * ROWS_PT, ROWS_TAIL)])

    return k(fts, cols2, rows_idx, ew2, zeros)


FBLK = 2000


def _csum_body(agg1_ref, b_ref, mskT_ref, out_ref):
    h1 = jnp.maximum(agg1_ref[...] + b_ref[...], 0.0)
    part = jax.lax.dot_general(h1, mskT_ref[...], (((0,), (0,)), ((), ())),
                               preferred_element_type=jnp.float32,
                               precision=_HIGH)

    @pl.when(pl.program_id(0) == 0)
    def _():
        out_ref[...] = part

    @pl.when(pl.program_id(0) > 0)
    def _():
        out_ref[...] += part


def _logits_body(csum_ref, mskT_ref, wd_ref, agg_ref, b_ref, sb_ref, bd_ref,
                 out_ref):
    cvec = csum_ref[...] / jnp.sum(mskT_ref[...])
    cvec = 1.0 / (1.0 + jnp.exp(-cvec))
    u = jax.lax.dot_general(wd_ref[...], cvec, (((1,), (0,)), ((), ())),
                            preferred_element_type=jnp.float32,
                            precision=_HIGH)
    h = jnp.maximum(agg_ref[...] + b_ref[...], 0.0)
    s = jax.lax.dot_general(h, u, (((1,), (0,)), ((), ())),
                            preferred_element_type=jnp.float32,
                            precision=_HIGH)
    out_ref[...] = s + bd_ref[0, 0] + sb_ref[...]


def _final(agg, b_gcn, mskT, wd, sb, bd):
    csum = pl.pallas_call(
        _csum_body,
        grid=(N // FBLK,),
        in_specs=[pl.BlockSpec((FBLK, NF), lambda i: (i, 0)),
                  pl.BlockSpec((1, NF), lambda i: (0, 0)),
                  pl.BlockSpec((FBLK, 1), lambda i: (i, 0))],
        out_specs=pl.BlockSpec((NF, 1), lambda i: (0, 0)),
        out_shape=jax.ShapeDtypeStruct((NF, 1), jnp.float32),
    )(agg[:N], b_gcn, mskT)
    return pl.pallas_call(
        _logits_body,
        grid=(2 * N // FBLK,),
        in_specs=[pl.BlockSpec((NF, 1), lambda i: (0, 0)),
                  pl.BlockSpec((N, 1), lambda i: (0, 0)),
                  pl.BlockSpec((NF, NF), lambda i: (0, 0)),
                  pl.BlockSpec((FBLK, NF), lambda i: (i, 0)),
                  pl.BlockSpec((1, NF), lambda i: (0, 0)),
                  pl.BlockSpec((FBLK, 1), lambda i: (i, 0)),
                  pl.BlockSpec((1, 1), lambda i: (0, 0))],
        out_specs=pl.BlockSpec((FBLK, 1), lambda i: (i, 0)),
        out_shape=jax.ShapeDtypeStruct((2 * N, 1), jnp.float32),
    )(csum, mskT, wd, agg, b_gcn, sb, bd)


def kernel(seq1, seq2, edge_index, edge_weight, msk, samp_bias1, samp_bias2,
           W_gcn, b_gcn, W_disc, b_disc):
    seqs = jnp.concatenate([seq1[0], seq2[0]], axis=0)
    fts = _dense_fts(seqs, W_gcn)

    row = edge_index[0]
    col = edge_index[1]
    pad = EPAD - E
    colp = jnp.concatenate([col, jnp.zeros((pad,), jnp.int32)])
    rowp = jnp.concatenate([row, jnp.zeros((pad,), jnp.int32)])
    ewp = jnp.concatenate([edge_weight, jnp.zeros((pad,), jnp.float32)])
    cols2 = jnp.stack([colp, colp + N]).reshape(NCORES, EPAD // SUB, SUB)
    rows_i = rowp.reshape(EPAD // SUB, SUB)
    ew2 = ewp.reshape(EPAD // 16, 16)
    zeros = jnp.zeros((ROWS_PT + ROWS_TAIL, NF), jnp.float32)

    agg = _sc_spmm(fts, cols2, rows_i, ew2, zeros)

    sb = jnp.concatenate([samp_bias1, samp_bias2], axis=1).reshape(2 * N, 1)
    out = _final(agg.reshape(2 * N, NF), b_gcn.reshape(1, NF),
                 msk.reshape(N, 1), W_disc, sb, b_disc.reshape(1, 1))
    return out.reshape(1, 2 * N)
